# baseline (device time: 14413 ns/iter reference)
import jax
import jax.numpy as jnp
from jax import lax
from jax.experimental import pallas as pl
from jax.experimental.pallas import tpu as pltpu

N_DEV = 32
N_PLANE = 8
N_LINE = 4

_sem_signal = getattr(pl, "semaphore_signal", None) or pltpu.semaphore_signal
_sem_wait = getattr(pl, "semaphore_wait", None) or pltpu.semaphore_wait
_CompilerParams = getattr(pltpu, "CompilerParams", None) or pltpu.TPUCompilerParams
_DeviceIdType = getattr(pl, "DeviceIdType", None) or pltpu.DeviceIdType


def kernel(x):
    m_per, n = x.shape

    def body(
        x_hbm,
        out_ref,
        xv_ref,
        comm1_ref,
        comm2_ref,
        send1_sems,
        recv1_sems,
        send2_sems,
        recv2_sems,
        load_sem,
    ):
        my_pos = lax.axis_index("i")
        my_z = my_pos // N_PLANE
        my_r = my_pos % N_PLANE

        load = pltpu.make_async_copy(x_hbm, xv_ref, load_sem)
        load.start()

        barrier_sem = pltpu.get_barrier_semaphore()
        _sem_signal(barrier_sem, inc=1)
        _sem_wait(barrier_sem, 1)

        load.wait()
        comm1_ref[pl.ds(my_r, 1), :] = jnp.max(
            xv_ref[:, :], axis=0, keepdims=True
        )

        sends1 = []
        for dr in range(1, N_PLANE):
            peer = my_z * N_PLANE + (my_r + dr) % N_PLANE
            s = pltpu.make_async_remote_copy(
                src_ref=comm1_ref.at[my_r],
                dst_ref=comm1_ref.at[my_r],
                send_sem=send1_sems.at[dr],
                recv_sem=recv1_sems.at[my_r],
                device_id=(peer,),
                device_id_type=_DeviceIdType.MESH,
            )
            s.start()
            sends1.append(s)

        for dr in range(1, N_PLANE):
            src_r = (my_r + dr) % N_PLANE
            recv = pltpu.make_async_remote_copy(
                src_ref=comm1_ref.at[src_r],
                dst_ref=comm1_ref.at[src_r],
                send_sem=send1_sems.at[dr],
                recv_sem=recv1_sems.at[src_r],
                device_id=(my_pos,),
                device_id_type=_DeviceIdType.MESH,
            )
            recv.wait_recv()

        comm2_ref[pl.ds(my_z, 1), :] = jnp.max(
            comm1_ref[:, :], axis=0, keepdims=True
        )

        sends2 = []
        for dz in range(1, N_LINE):
            peer = ((my_z + dz) % N_LINE) * N_PLANE + my_r
            s = pltpu.make_async_remote_copy(
                src_ref=comm2_ref.at[my_z],
                dst_ref=comm2_ref.at[my_z],
                send_sem=send2_sems.at[dz],
                recv_sem=recv2_sems.at[my_z],
                device_id=(peer,),
                device_id_type=_DeviceIdType.MESH,
            )
            s.start()
            sends2.append(s)

        for dz in range(1, N_LINE):
            src_z = (my_z + dz) % N_LINE
            recv = pltpu.make_async_remote_copy(
                src_ref=comm2_ref.at[src_z],
                dst_ref=comm2_ref.at[src_z],
                send_sem=send2_sems.at[dz],
                recv_sem=recv2_sems.at[src_z],
                device_id=(my_pos,),
                device_id_type=_DeviceIdType.MESH,
            )
            recv.wait_recv()

        out_ref[:, :] = jnp.max(comm2_ref[:, :], axis=0, keepdims=True)

        for s in sends1 + sends2:
            s.wait_send()

    return pl.pallas_call(
        body,
        out_shape=jax.ShapeDtypeStruct((1, n), x.dtype),
        in_specs=[pl.BlockSpec(memory_space=pl.ANY)],
        out_specs=pl.BlockSpec(memory_space=pltpu.VMEM),
        scratch_shapes=[
            pltpu.VMEM((m_per, n), x.dtype),
            pltpu.VMEM((N_PLANE, n), x.dtype),
            pltpu.VMEM((N_LINE, n), x.dtype),
            pltpu.SemaphoreType.DMA((N_PLANE,)),
            pltpu.SemaphoreType.DMA((N_PLANE,)),
            pltpu.SemaphoreType.DMA((N_LINE,)),
            pltpu.SemaphoreType.DMA((N_LINE,)),
            pltpu.SemaphoreType.DMA,
        ],
        compiler_params=_CompilerParams(collective_id=0),
    )(x)


# device time: 14067 ns/iter; 1.0246x vs baseline; 1.0246x over previous
import jax
import jax.numpy as jnp
from jax import lax
from jax.experimental import pallas as pl
from jax.experimental.pallas import tpu as pltpu

N_DEV = 32

_sem_signal = getattr(pl, "semaphore_signal", None) or pltpu.semaphore_signal
_sem_wait = getattr(pl, "semaphore_wait", None) or pltpu.semaphore_wait
_CompilerParams = getattr(pltpu, "CompilerParams", None) or pltpu.TPUCompilerParams
_DeviceIdType = getattr(pl, "DeviceIdType", None) or pltpu.DeviceIdType

_SEND_ORDER = sorted(range(1, N_DEV), key=lambda d: -min(d, N_DEV - d))
_RECV_ORDER = sorted(range(1, N_DEV), key=lambda d: min(d, N_DEV - d))


def kernel(x):
    m_per, n = x.shape

    def body(x_ref, out_ref, comm_ref, res_ref, send_sems, recv_sems, out_sem):
        my_pos = lax.axis_index("i")

        barrier_sem = pltpu.get_barrier_semaphore()
        _sem_signal(barrier_sem, inc=1)
        _sem_wait(barrier_sem, 1)

        comm_ref[pl.ds(my_pos, 1), :] = jnp.max(
            x_ref[:, :], axis=0, keepdims=True
        )

        sends = []
        for d in _SEND_ORDER:
            s = pltpu.make_async_remote_copy(
                src_ref=comm_ref.at[my_pos],
                dst_ref=comm_ref.at[my_pos],
                send_sem=send_sems.at[d],
                recv_sem=recv_sems.at[my_pos],
                device_id=((my_pos + d) % N_DEV,),
                device_id_type=_DeviceIdType.MESH,
            )
            s.start()
            sends.append(s)

        for d in _RECV_ORDER:
            src_pos = (my_pos + d) % N_DEV
            recv = pltpu.make_async_remote_copy(
                src_ref=comm_ref.at[src_pos],
                dst_ref=comm_ref.at[src_pos],
                send_sem=send_sems.at[d],
                recv_sem=recv_sems.at[src_pos],
                device_id=(my_pos,),
                device_id_type=_DeviceIdType.MESH,
            )
            recv.wait_recv()

        res_ref[:, :] = jnp.max(comm_ref[:, :], axis=0, keepdims=True)
        out_cp = pltpu.make_async_copy(res_ref, out_ref, out_sem)
        out_cp.start()
        out_cp.wait()

        for s in sends:
            s.wait_send()

    return pl.pallas_call(
        body,
        out_shape=jax.ShapeDtypeStruct((1, n), x.dtype),
        in_specs=[pl.BlockSpec(memory_space=pltpu.MemorySpace.VMEM)],
        out_specs=pl.BlockSpec(memory_space=pl.ANY),
        scratch_shapes=[
            pltpu.VMEM((N_DEV, n), x.dtype),
            pltpu.VMEM((1, n), x.dtype),
            pltpu.SemaphoreType.DMA((N_DEV,)),
            pltpu.SemaphoreType.DMA((N_DEV,)),
            pltpu.SemaphoreType.DMA,
        ],
        compiler_params=_CompilerParams(collective_id=0),
    )(x)
